# Initial kernel scaffold; baseline (speedup 1.0000x reference)
import jax
import jax.numpy as jnp
from jax.experimental import pallas as pl


def kernel(x_node, x0, x2, x3, edge_index_1, edge_index_2, edge_index_12, edge_index_13, edge_weight_1, edge_weight_2, W_s1s, b_s1s, W_s2s, b_s2s, W_s121s, b_s121s, W_s131s, b_s131s, att_vec):
    raise NotImplementedError("write your pallas kernel here")


# speedup vs baseline: 6.0838x; 6.0838x over previous
import jax
import jax.numpy as jnp
from jax import lax
from jax.experimental import pallas as pl
from jax.experimental.pallas import tpu as pltpu
from jax.experimental.pallas import tpu_sc as plsc

NC = 2
NS = 16
NW = NC * NS
LANES = 16
CH = 125


def _mesh():
    return plsc.VectorSubcoreMesh(core_axis_name="c", subcore_axis_name="s",
                                  num_cores=NC, num_subcores=NS)


def _sc_pass(src, gidx2d, sidx2d, w2d, zeros):
    D = src.shape[1]
    n_pad = zeros.shape[0]
    nchunks = gidx2d.shape[0]
    rows_per_tile = nchunks // NW
    n_per_tile = n_pad // NS
    weighted = w2d is not None
    n_full = CH // LANES
    tail = CH - n_full * LANES

    def body(*refs):
        if weighted:
            (src_h, gidx_h, sidx_h, w_h, zero_h, out_h,
             acc, gidx_v, sidx_v, w_v, rows_v, sem) = refs
        else:
            (src_h, gidx_h, sidx_h, zero_h, out_h,
             acc, gidx_v, sidx_v, rows_v, sem) = refs
        cid = lax.axis_index("c")
        sid = lax.axis_index("s")
        wid = cid * NS + sid
        row0 = sid * n_per_tile
        pltpu.sync_copy(zero_h.at[pl.ds(row0, n_per_tile)],
                        acc.at[pl.ds(row0, n_per_tile)])
        base = wid * rows_per_tile
        pltpu.sync_copy(gidx_h.at[pl.ds(base, rows_per_tile)], gidx_v)
        pltpu.sync_copy(sidx_h.at[pl.ds(base, rows_per_tile)], sidx_v)
        if weighted:
            pltpu.sync_copy(w_h.at[pl.ds(base, rows_per_tile)], w_v)
        plsc.subcore_barrier()

        def mul_rows(j, base_row, ks):
            wv = w_v[j, pl.ds(base_row, LANES)]
            for k in ks:
                wk = wv[k]
                row = base_row + k
                for t in range(D // LANES):
                    sl = pl.ds(t * LANES, LANES)
                    rows_v[row, sl] = rows_v[row, sl] * wk

        def step(j, carry):
            pltpu.async_copy(src_h.at[gidx_v.at[j]], rows_v, sem).wait()
            if weighted:
                def wstep(g, c2):
                    mul_rows(j, g * LANES, range(LANES))
                    return c2
                lax.fori_loop(0, n_full, wstep, 0)
                if tail:
                    mul_rows(j, CH - LANES, range(LANES - tail, LANES))
            pltpu.sync_copy(rows_v, acc.at[sidx_v.at[j]], add=True)
            return carry

        lax.fori_loop(0, rows_per_tile, step, 0)
        plsc.subcore_barrier()
        pltpu.sync_copy(acc.at[pl.ds(row0, n_per_tile)],
                        out_h.at[cid, pl.ds(row0, n_per_tile)])

    scratch = [pltpu.VMEM_SHARED((n_pad, D), jnp.float32),
               pltpu.VMEM((rows_per_tile, CH), jnp.int32),
               pltpu.VMEM((rows_per_tile, CH), jnp.int32)]
    if weighted:
        scratch.append(pltpu.VMEM((rows_per_tile, CH), jnp.float32))
    scratch += [pltpu.VMEM((CH, D), jnp.float32), pltpu.SemaphoreType.DMA]

    fn = pl.kernel(body,
                   out_type=jax.ShapeDtypeStruct((NC, n_pad, D), jnp.float32),
                   mesh=_mesh(), scratch_types=scratch)
    args = (src, gidx2d, sidx2d) + ((w2d,) if weighted else ()) + (zeros,)
    return fn(*args)


def _sc_counts(idx2ds, n_pad):
    nidx = len(idx2ds)
    nchunks = idx2ds[0].shape[0]
    rows_per_tile = nchunks // NW
    cols_per_tile = n_pad // NS

    def body(*refs):
        idx_hs = refs[:nidx]
        out_h = refs[nidx]
        accs = refs[nidx + 1:nidx + 1 + nidx]
        idx_v, ones_v, zbuf = refs[nidx + 1 + nidx:]
        cid = lax.axis_index("c")
        sid = lax.axis_index("s")
        wid = cid * NS + sid
        col0 = sid * cols_per_tile

        def zstep(i, c):
            zbuf[pl.ds(i * LANES, LANES)] = jnp.zeros((LANES,), jnp.float32)
            return c
        lax.fori_loop(0, cols_per_tile // LANES, zstep, 0)
        for t in range(128 // LANES):
            ones_v[pl.ds(t * LANES, LANES)] = jnp.ones((LANES,), jnp.float32)
        for a in range(nidx):
            pltpu.sync_copy(zbuf, accs[a].at[pl.ds(col0, cols_per_tile)])
        plsc.subcore_barrier()

        base = wid * rows_per_tile
        for a in range(nidx):
            pltpu.sync_copy(idx_hs[a].at[pl.ds(base, rows_per_tile)], idx_v)

            def step(j, carry, a=a):
                pltpu.sync_copy(ones_v.at[pl.ds(0, CH)],
                                accs[a].at[idx_v.at[j]], add=True)
                return carry
            lax.fori_loop(0, rows_per_tile, step, 0)
        plsc.subcore_barrier()
        for a in range(nidx):
            pltpu.sync_copy(accs[a].at[pl.ds(col0, cols_per_tile)],
                            out_h.at[cid, a, pl.ds(col0, cols_per_tile)])

    scratch = ([pltpu.VMEM_SHARED((n_pad,), jnp.float32)] * nidx +
               [pltpu.VMEM((rows_per_tile, CH), jnp.int32),
                pltpu.VMEM((128,), jnp.float32),
                pltpu.VMEM((cols_per_tile,), jnp.float32)])
    fn = pl.kernel(body,
                   out_type=jax.ShapeDtypeStruct((NC, nidx, n_pad), jnp.float32),
                   mesh=_mesh(), scratch_types=scratch)
    return fn(*idx2ds)


def _tc_inv(cnt_partial):
    nidx, n_pad = cnt_partial.shape[1], cnt_partial.shape[2]

    def body(c_ref, o_ref):
        o_ref[...] = 1.0 / jnp.maximum(c_ref[0] + c_ref[1], 1.0)

    return pl.pallas_call(
        body,
        out_shape=jax.ShapeDtypeStruct((nidx, n_pad), jnp.float32),
    )(cnt_partial)


def _tc_combine_avg(p, inv, x):
    n_pad, D = x.shape
    B = 1024

    def body(p_ref, inv_ref, x_ref, o_ref):
        m = (p_ref[0] + p_ref[1]) * inv_ref[...]
        o_ref[...] = (m + x_ref[...]) * 0.5

    return pl.pallas_call(
        body,
        grid=(n_pad // B,),
        in_specs=[pl.BlockSpec((2, B, D), lambda i: (0, i, 0)),
                  pl.BlockSpec((B, 1), lambda i: (i, 0)),
                  pl.BlockSpec((B, D), lambda i: (i, 0))],
        out_specs=pl.BlockSpec((B, D), lambda i: (i, 0)),
        out_shape=jax.ShapeDtypeStruct((n_pad, D), jnp.float32),
    )(p, inv, x)


def _tc_final(p1, p2, p3, p4, inv1, inv2, W1t, b1, W2t, b2, W3t, b3, W4t, b4,
              av):
    n_pad, D = p1.shape[1], p1.shape[2]
    B = 1024

    def body(p1_ref, p2_ref, p3_ref, p4_ref, inv1_ref, inv2_ref,
             W1_ref, b1_ref, W2_ref, b2_ref, W3_ref, b3_ref, W4_ref, b4_ref,
             av_ref, o_ref):
        def head(p_ref, inv_ref, W_ref, b_ref):
            pre = (p_ref[0] + p_ref[1]) * inv_ref[...]
            h = jnp.dot(pre, W_ref[...], preferred_element_type=jnp.float32)
            return jnp.maximum(h + b_ref[...], 0.0)

        a1 = head(p1_ref, inv1_ref, W1_ref, b1_ref)
        a2 = head(p2_ref, inv2_ref, W2_ref, b2_ref)
        a3 = head(p3_ref, inv1_ref, W3_ref, b3_ref)
        a4 = head(p4_ref, inv1_ref, W4_ref, b4_ref)
        av = av_ref[...]
        s1 = jnp.sum(a1 * av[0:1, :], axis=1, keepdims=True)
        s2 = jnp.sum(a2 * av[1:2, :], axis=1, keepdims=True)
        s3 = jnp.sum(a3 * av[2:3, :], axis=1, keepdims=True)
        s4 = jnp.sum(a4 * av[3:4, :], axis=1, keepdims=True)
        m = jnp.maximum(jnp.maximum(s1, s2), jnp.maximum(s3, s4))
        e1 = jnp.exp(s1 - m)
        e2 = jnp.exp(s2 - m)
        e3 = jnp.exp(s3 - m)
        e4 = jnp.exp(s4 - m)
        z = e1 + e2 + e3 + e4
        o_ref[...] = (e1 * a1 + e2 * a2 + e3 * a3 + e4 * a4) / z

    pspec = pl.BlockSpec((2, B, D), lambda i: (0, i, 0))
    ispec = pl.BlockSpec((B, 1), lambda i: (i, 0))
    wspec = pl.BlockSpec((D, D), lambda i: (0, 0))
    bspec = pl.BlockSpec((1, D), lambda i: (0, 0))
    return pl.pallas_call(
        body,
        grid=(n_pad // B,),
        in_specs=[pspec, pspec, pspec, pspec, ispec, ispec,
                  wspec, bspec, wspec, bspec, wspec, bspec, wspec, bspec,
                  pl.BlockSpec((4, D), lambda i: (0, 0))],
        out_specs=pl.BlockSpec((B, D), lambda i: (i, 0)),
        out_shape=jax.ShapeDtypeStruct((n_pad, D), jnp.float32),
    )(p1, p2, p3, p4, inv1, inv2, W1t, b1, W2t, b2, W3t, b3, W4t, b4, av)


def kernel(x_node, x0, x2, x3, edge_index_1, edge_index_2, edge_index_12,
           edge_index_13, edge_weight_1, edge_weight_2, W_s1s, b_s1s,
           W_s2s, b_s2s, W_s121s, b_s121s, W_s131s, b_s131s, att_vec):
    N, D = x_node.shape
    E = edge_index_1.shape[1]
    n_pad = ((N + (NS * LANES) - 1) // (NS * LANES)) * (NS * LANES)

    def chunk_i(v):
        return v.reshape(E // CH, CH)

    def padn(v):
        return jnp.concatenate(
            [v, jnp.zeros((n_pad - N, D), jnp.float32)], axis=0)

    e1s, e1d = chunk_i(edge_index_1[0]), chunk_i(edge_index_1[1])
    e2s, e2d = chunk_i(edge_index_2[0]), chunk_i(edge_index_2[1])
    e12s, e12d = chunk_i(edge_index_12[0]), chunk_i(edge_index_12[1])
    e13s, e13d = chunk_i(edge_index_13[0]), chunk_i(edge_index_13[1])
    w1 = edge_weight_1.reshape(E // CH, CH)
    w2 = edge_weight_2.reshape(E // CH, CH)
    zeros = jnp.zeros((n_pad, D), jnp.float32)
    x0p, x2p, x3p = padn(x0), padn(x2), padn(x3)

    cnt_p = _sc_counts([e1d, e1s, e2d, e2s, e12d, e12s, e13d, e13s], n_pad)
    inv8 = _tc_inv(cnt_p)

    def inv(i):
        return inv8[i].reshape(n_pad, 1)

    inv_e1d, inv_e1s = inv(0), inv(1)
    inv_e2d, inv_e2s = inv(2), inv(3)
    inv_e12d, inv_e12s = inv(4), inv(5)
    inv_e13d, inv_e13s = inv(6), inv(7)

    n1 = _tc_combine_avg(_sc_pass(x_node, e1s, e1d, w1, zeros), inv_e1d, x0p)
    p_s1 = _sc_pass(n1, e1d, e1s, None, zeros)
    n_2 = _tc_combine_avg(_sc_pass(x_node, e2s, e2d, w2, zeros), inv_e2d, x2p)
    p_s2 = _sc_pass(n_2, e2d, e2s, None, zeros)
    n2 = _tc_combine_avg(_sc_pass(n1, e12s, e12d, None, zeros), inv_e12d, x2p)
    n3 = _tc_combine_avg(_sc_pass(n2, e12d, e12s, None, zeros), inv_e12s, x0p)
    p_s121 = _sc_pass(n3, e1d, e1s, w1, zeros)
    n2b = _tc_combine_avg(_sc_pass(n1, e13s, e13d, None, zeros), inv_e13d, x3p)
    n3b = _tc_combine_avg(_sc_pass(n2b, e13d, e13s, None, zeros), inv_e13s, x0p)
    p_s131 = _sc_pass(n3b, e1d, e1s, w1, zeros)

    h = _tc_final(p_s1, p_s2, p_s121, p_s131, inv_e1s, inv_e2s,
                  W_s1s.T, b_s1s.reshape(1, D), W_s2s.T, b_s2s.reshape(1, D),
                  W_s121s.T, b_s121s.reshape(1, D), W_s131s.T,
                  b_s131s.reshape(1, D), att_vec)
    return h[:N]


# speedup vs baseline: 7.9148x; 1.3010x over previous
import jax
import jax.numpy as jnp
from jax import lax
from jax.experimental import pallas as pl
from jax.experimental.pallas import tpu as pltpu
from jax.experimental.pallas import tpu_sc as plsc

NC = 2
NS = 16
NW = NC * NS
LANES = 16
CH = 125


def _mesh():
    return plsc.VectorSubcoreMesh(core_axis_name="c", subcore_axis_name="s",
                                  num_cores=NC, num_subcores=NS)


def _sc_pass(src, packed, w2d, zeros):
    D = src.shape[1]
    n_pad = zeros.shape[0]
    nchunks, K, _ = packed.shape
    rows_per_tile = nchunks // NW
    n_per_tile = n_pad // NS
    n_full = CH // LANES
    tail = CH - n_full * LANES
    n_outer = rows_per_tile // 2
    weighted = w2d is not None

    def body(*refs):
        if weighted:
            (src_h, pk_h, w_h, zero_h, out_h,
             acc, ib0, ib1, wb0, wb1, r0, r1) = refs[:12]
            sems = refs[12:]
            wbufs = (wb0, wb1)
            wsems = sems[6:8]
        else:
            (src_h, pk_h, zero_h, out_h, acc, ib0, ib1, r0, r1) = refs[:9]
            sems = refs[9:]
        ibufs = (ib0, ib1)
        rows = (r0, r1)
        isems, gsems, ssems = sems[0:2], sems[2:4], sems[4:6]
        cid = lax.axis_index("c")
        sid = lax.axis_index("s")
        wid = cid * NS + sid
        row0 = sid * n_per_tile
        base = wid * rows_per_tile
        pltpu.sync_copy(zero_h.at[pl.ds(row0, n_per_tile)],
                        acc.at[pl.ds(row0, n_per_tile)])

        def istart(b, j):
            pltpu.async_copy(pk_h.at[base + j], ibufs[b], isems[b])
            if weighted:
                pltpu.async_copy(w_h.at[base + j], wbufs[b], wsems[b])

        def iwait(b, j):
            pltpu.make_async_copy(pk_h.at[base + j], ibufs[b],
                                  isems[b]).wait()
            if weighted:
                pltpu.make_async_copy(w_h.at[base + j], wbufs[b],
                                      wsems[b]).wait()

        def gstart(b):
            pltpu.async_copy(src_h.at[ibufs[b].at[0]], rows[b], gsems[b])

        def gwait(b):
            pltpu.make_async_copy(src_h.at[ibufs[b].at[0]], rows[b],
                                  gsems[b]).wait()

        def sstart(b):
            pltpu.async_copy(rows[b], acc.at[ibufs[b].at[1]], ssems[b],
                             add=True)

        def swait(b):
            pltpu.make_async_copy(rows[b], acc.at[ibufs[b].at[1]],
                                  ssems[b]).wait()

        def mul_rows(b, base_row, ks):
            wv = wbufs[b][pl.ds(base_row, LANES)]
            for k in ks:
                wk = wv[k]
                row = base_row + k
                for t in range(D // LANES):
                    sl = pl.ds(t * LANES, LANES)
                    rows[b][row, sl] = rows[b][row, sl] * wk

        def mul_chunk(b):
            def wstep(g, c2):
                mul_rows(b, g * LANES, range(LANES))
                return c2
            lax.fori_loop(0, n_full, wstep, 0)
            if tail:
                mul_rows(b, CH - LANES, range(LANES - tail, LANES))

        istart(0, 0)
        istart(1, 1)
        plsc.subcore_barrier()
        iwait(0, 0)
        gstart(0)

        def outer(go, carry):
            j0 = 2 * go
            iwait(1, j0 + 1)
            gstart(1)
            gwait(0)
            if weighted:
                mul_chunk(0)
            sstart(0)
            gwait(1)
            if weighted:
                mul_chunk(1)
            sstart(1)
            swait(0)

            @pl.when(go != n_outer - 1)
            def _refill0():
                istart(0, j0 + 2)
                iwait(0, j0 + 2)
                gstart(0)
            swait(1)

            @pl.when(go != n_outer - 1)
            def _refill1():
                istart(1, j0 + 3)
            return carry

        lax.fori_loop(0, n_outer, outer, 0)
        plsc.subcore_barrier()
        pltpu.sync_copy(acc.at[pl.ds(row0, n_per_tile)],
                        out_h.at[cid, pl.ds(row0, n_per_tile)])

    scratch = [pltpu.VMEM_SHARED((n_pad, D), jnp.float32)]
    scratch += [pltpu.VMEM((K, CH), jnp.int32) for _ in range(2)]
    if weighted:
        scratch += [pltpu.VMEM((CH,), jnp.float32) for _ in range(2)]
    scratch += [pltpu.VMEM((CH, D), jnp.float32) for _ in range(2)]
    scratch += [pltpu.SemaphoreType.DMA for _ in range(8 if weighted else 6)]
    fn = pl.kernel(body,
                   out_type=jax.ShapeDtypeStruct((NC, n_pad, D), jnp.float32),
                   mesh=_mesh(), scratch_types=scratch)
    args = (src, packed) + ((w2d,) if weighted else ()) + (zeros,)
    return fn(*args)


def _sc_counts(idx2ds, n_pad):
    nidx = len(idx2ds)
    nchunks = idx2ds[0].shape[0]
    rows_per_tile = nchunks // NW
    cols_per_tile = n_pad // NS

    def body(*refs):
        idx_hs = refs[:nidx]
        out_h = refs[nidx]
        accs = refs[nidx + 1:nidx + 1 + nidx]
        idx_v, ones_v, zbuf = refs[nidx + 1 + nidx:]
        cid = lax.axis_index("c")
        sid = lax.axis_index("s")
        wid = cid * NS + sid
        col0 = sid * cols_per_tile

        def zstep(i, c):
            zbuf[pl.ds(i * LANES, LANES)] = jnp.zeros((LANES,), jnp.float32)
            return c
        lax.fori_loop(0, cols_per_tile // LANES, zstep, 0)
        for t in range(128 // LANES):
            ones_v[pl.ds(t * LANES, LANES)] = jnp.ones((LANES,), jnp.float32)
        for a in range(nidx):
            pltpu.sync_copy(zbuf, accs[a].at[pl.ds(col0, cols_per_tile)])
        plsc.subcore_barrier()

        base = wid * rows_per_tile
        for a in range(nidx):
            pltpu.sync_copy(idx_hs[a].at[pl.ds(base, rows_per_tile)], idx_v)

            def step(j, carry, a=a):
                pltpu.sync_copy(ones_v.at[pl.ds(0, CH)],
                                accs[a].at[idx_v.at[j]], add=True)
                return carry
            lax.fori_loop(0, rows_per_tile, step, 0)
        plsc.subcore_barrier()
        for a in range(nidx):
            pltpu.sync_copy(accs[a].at[pl.ds(col0, cols_per_tile)],
                            out_h.at[cid, a, pl.ds(col0, cols_per_tile)])

    scratch = ([pltpu.VMEM_SHARED((n_pad,), jnp.float32)] * nidx +
               [pltpu.VMEM((rows_per_tile, CH), jnp.int32),
                pltpu.VMEM((128,), jnp.float32),
                pltpu.VMEM((cols_per_tile,), jnp.float32)])
    fn = pl.kernel(body,
                   out_type=jax.ShapeDtypeStruct((NC, nidx, n_pad), jnp.float32),
                   mesh=_mesh(), scratch_types=scratch)
    return fn(*idx2ds)


def _tc_inv(cnt_partial):
    nidx, n_pad = cnt_partial.shape[1], cnt_partial.shape[2]

    def body(c_ref, o_ref):
        o_ref[...] = 1.0 / jnp.maximum(c_ref[0] + c_ref[1], 1.0)

    return pl.pallas_call(
        body,
        out_shape=jax.ShapeDtypeStruct((nidx, n_pad), jnp.float32),
    )(cnt_partial)


def _tc_combine_avg(p, inv, x):
    n_pad, D = x.shape
    B = 1024

    def body(p_ref, inv_ref, x_ref, o_ref):
        m = (p_ref[0] + p_ref[1]) * inv_ref[...]
        o_ref[...] = (m + x_ref[...]) * 0.5

    return pl.pallas_call(
        body,
        grid=(n_pad // B,),
        in_specs=[pl.BlockSpec((2, B, D), lambda i: (0, i, 0)),
                  pl.BlockSpec((B, 1), lambda i: (i, 0)),
                  pl.BlockSpec((B, D), lambda i: (i, 0))],
        out_specs=pl.BlockSpec((B, D), lambda i: (i, 0)),
        out_shape=jax.ShapeDtypeStruct((n_pad, D), jnp.float32),
    )(p, inv, x)


def _tc_final(p1, p2, p3, p4, inv1, inv2, W1t, b1, W2t, b2, W3t, b3, W4t, b4,
              av):
    n_pad, D = p1.shape[1], p1.shape[2]
    B = 1024

    def body(p1_ref, p2_ref, p3_ref, p4_ref, inv1_ref, inv2_ref,
             W1_ref, b1_ref, W2_ref, b2_ref, W3_ref, b3_ref, W4_ref, b4_ref,
             av_ref, o_ref):
        def head(p_ref, inv_ref, W_ref, b_ref):
            pre = (p_ref[0] + p_ref[1]) * inv_ref[...]
            h = jnp.dot(pre, W_ref[...], preferred_element_type=jnp.float32)
            return jnp.maximum(h + b_ref[...], 0.0)

        a1 = head(p1_ref, inv1_ref, W1_ref, b1_ref)
        a2 = head(p2_ref, inv2_ref, W2_ref, b2_ref)
        a3 = head(p3_ref, inv1_ref, W3_ref, b3_ref)
        a4 = head(p4_ref, inv1_ref, W4_ref, b4_ref)
        av = av_ref[...]
        s1 = jnp.sum(a1 * av[0:1, :], axis=1, keepdims=True)
        s2 = jnp.sum(a2 * av[1:2, :], axis=1, keepdims=True)
        s3 = jnp.sum(a3 * av[2:3, :], axis=1, keepdims=True)
        s4 = jnp.sum(a4 * av[3:4, :], axis=1, keepdims=True)
        m = jnp.maximum(jnp.maximum(s1, s2), jnp.maximum(s3, s4))
        e1 = jnp.exp(s1 - m)
        e2 = jnp.exp(s2 - m)
        e3 = jnp.exp(s3 - m)
        e4 = jnp.exp(s4 - m)
        z = e1 + e2 + e3 + e4
        o_ref[...] = (e1 * a1 + e2 * a2 + e3 * a3 + e4 * a4) / z

    pspec = pl.BlockSpec((2, B, D), lambda i: (0, i, 0))
    ispec = pl.BlockSpec((B, 1), lambda i: (i, 0))
    wspec = pl.BlockSpec((D, D), lambda i: (0, 0))
    bspec = pl.BlockSpec((1, D), lambda i: (0, 0))
    return pl.pallas_call(
        body,
        grid=(n_pad // B,),
        in_specs=[pspec, pspec, pspec, pspec, ispec, ispec,
                  wspec, bspec, wspec, bspec, wspec, bspec, wspec, bspec,
                  pl.BlockSpec((4, D), lambda i: (0, 0))],
        out_specs=pl.BlockSpec((B, D), lambda i: (i, 0)),
        out_shape=jax.ShapeDtypeStruct((n_pad, D), jnp.float32),
    )(p1, p2, p3, p4, inv1, inv2, W1t, b1, W2t, b2, W3t, b3, W4t, b4, av)


def kernel(x_node, x0, x2, x3, edge_index_1, edge_index_2, edge_index_12,
           edge_index_13, edge_weight_1, edge_weight_2, W_s1s, b_s1s,
           W_s2s, b_s2s, W_s121s, b_s121s, W_s131s, b_s131s, att_vec):
    N, D = x_node.shape
    E = edge_index_1.shape[1]
    n_pad = ((N + (NS * LANES) - 1) // (NS * LANES)) * (NS * LANES)

    def chunk_i(v):
        return v.reshape(E // CH, CH)

    def padn(v):
        return jnp.concatenate(
            [v, jnp.zeros((n_pad - N, D), jnp.float32)], axis=0)

    e1s, e1d = chunk_i(edge_index_1[0]), chunk_i(edge_index_1[1])
    e2s, e2d = chunk_i(edge_index_2[0]), chunk_i(edge_index_2[1])
    e12s, e12d = chunk_i(edge_index_12[0]), chunk_i(edge_index_12[1])
    e13s, e13d = chunk_i(edge_index_13[0]), chunk_i(edge_index_13[1])
    w1 = edge_weight_1.reshape(E // CH, CH)
    w2 = edge_weight_2.reshape(E // CH, CH)

    def pack(g, s):
        return jnp.stack([g, s], axis=1)

    pk_m1 = pack(e1s, e1d)
    pk_s1 = pack(e1d, e1s)
    pk_s121 = pack(e1d, e1s)
    pk_m2 = pack(e2s, e2d)
    pk_s2 = pack(e2d, e2s)
    pk_e12f, pk_e12b = pack(e12s, e12d), pack(e12d, e12s)
    pk_e13f, pk_e13b = pack(e13s, e13d), pack(e13d, e13s)
    zeros = jnp.zeros((n_pad, D), jnp.float32)
    x0p, x2p, x3p = padn(x0), padn(x2), padn(x3)

    cnt_p = _sc_counts([e1d, e1s, e2d, e2s, e12d, e12s, e13d, e13s], n_pad)
    inv8 = _tc_inv(cnt_p)

    def inv(i):
        return inv8[i].reshape(n_pad, 1)

    inv_e1d, inv_e1s = inv(0), inv(1)
    inv_e2d, inv_e2s = inv(2), inv(3)
    inv_e12d, inv_e12s = inv(4), inv(5)
    inv_e13d, inv_e13s = inv(6), inv(7)

    n1 = _tc_combine_avg(_sc_pass(x_node, pk_m1, w1, zeros), inv_e1d, x0p)
    p_s1 = _sc_pass(n1, pk_s1, None, zeros)
    n_2 = _tc_combine_avg(_sc_pass(x_node, pk_m2, w2, zeros), inv_e2d, x2p)
    p_s2 = _sc_pass(n_2, pk_s2, None, zeros)
    n2 = _tc_combine_avg(_sc_pass(n1, pk_e12f, None, zeros), inv_e12d, x2p)
    n3 = _tc_combine_avg(_sc_pass(n2, pk_e12b, None, zeros), inv_e12s, x0p)
    p_s121 = _sc_pass(n3, pk_s121, w1, zeros)
    n2b = _tc_combine_avg(_sc_pass(n1, pk_e13f, None, zeros), inv_e13d, x3p)
    n3b = _tc_combine_avg(_sc_pass(n2b, pk_e13b, None, zeros), inv_e13s, x0p)
    p_s131 = _sc_pass(n3b, pk_s121, w1, zeros)

    h = _tc_final(p_s1, p_s2, p_s121, p_s131, inv_e1s, inv_e2s,
                  W_s1s.T, b_s1s.reshape(1, D), W_s2s.T, b_s2s.reshape(1, D),
                  W_s121s.T, b_s121s.reshape(1, D), W_s131s.T,
                  b_s131s.reshape(1, D), att_vec)
    return h[:N]
